# baseline (speedup 1.0000x reference)
import functools

import jax
import jax.numpy as jnp
from jax import lax
from jax.experimental import pallas as pl
from jax.experimental.pallas import tpu as pltpu
from jax.experimental.pallas import tpu_sc as plsc

_BLK = 128


def _make_sc_decode(N_pad, A, P, NC, NW, n_iters):
    A_pad = ((A + 15) // 16) * 16
    mesh = plsc.VectorSubcoreMesh(core_axis_name="c", subcore_axis_name="s")
    tile_words = P * _BLK

    @functools.partial(
        pl.kernel,
        mesh=mesh,
        out_type=jax.ShapeDtypeStruct((P, N_pad), jnp.float32),
        scratch_types=[
            pltpu.VMEM((_BLK,), jnp.int32),
            pltpu.VMEM((A_pad,), jnp.int32),
            pltpu.VMEM((A, P, _BLK), jnp.float32),
            pltpu.VMEM((P, _BLK), jnp.float32),
            pltpu.SemaphoreType.DMA,
            pltpu.SemaphoreType.DMA,
        ],
        compiler_params=pltpu.CompilerParams(
            use_tc_tiling_on_sc=True, needs_layout_passes=False
        ),
    )
    def k(table_hbm, acts_hbm, out_hbm, acts_v, present_v, buf,
          rows_v, sem, sem_out):
        wid = lax.axis_index("s") * NC + lax.axis_index("c")
        lane = lax.iota(jnp.int32, 16)
        zeros16 = jnp.zeros((16,), jnp.int32)
        ones16 = jnp.ones((16,), jnp.int32)

        def do_block(kb, _):
            b = pl.multiple_of((kb * NW + wid) * _BLK, _BLK)

            @pl.when(b < N_pad)
            def _process():
                pltpu.sync_copy(acts_hbm.at[pl.ds(b, _BLK)], acts_v)
                for ch in range(A_pad // 16):
                    present_v[pl.ds(ch * 16, 16)] = zeros16
                for ch in range(_BLK // 16):
                    a16 = acts_v[pl.ds(ch * 16, 16)]
                    plsc.store_scatter(present_v, [a16], ones16)
                pres = [
                    present_v[pl.ds(ch * 16, 16)] for ch in range(A_pad // 16)
                ]

                for a in range(A):
                    @pl.when(pres[a // 16][a % 16] != 0)
                    def _go(a=a):
                        pltpu.async_copy(
                            table_hbm.at[a, :, pl.ds(b, _BLK)],
                            buf.at[a],
                            sem,
                        )

                acc = pres[0]
                for ch in range(1, A_pad // 16):
                    acc = acc + pres[ch]
                n_tiles = jnp.sum(acc)

                def drain(_i, _c):
                    pltpu.make_async_copy(
                        table_hbm.at[0, :, pl.ds(b, _BLK)],
                        buf.at[0],
                        sem,
                    ).wait()
                    return _c

                lax.fori_loop(0, n_tiles, drain, 0)

                for ch in range(_BLK // 16):
                    a16 = acts_v[pl.ds(ch * 16, 16)]
                    l16 = ch * 16 + lane
                    for p in range(P):
                        vals = plsc.load_gather(
                            buf, [a16, jnp.full((16,), p, jnp.int32), l16]
                        )
                        rows_v[p, pl.ds(ch * 16, 16)] = vals
                pltpu.async_copy(
                    rows_v, out_hbm.at[:, pl.ds(b, _BLK)], sem_out
                ).wait()

            return _

        lax.fori_loop(0, n_iters, do_block, 0)

    return k


def kernel(abs_actions, decode_map):
    N, A, P = decode_map.shape
    info = plsc.get_sparse_core_info()
    NC = info.num_cores
    NW = NC * info.num_subcores

    N_pad = ((N + _BLK - 1) // _BLK) * _BLK
    NB = N_pad // _BLK
    n_iters = (NB + NW - 1) // NW

    table = decode_map.transpose(1, 2, 0)
    acts = jnp.pad(abs_actions.astype(jnp.int32), (0, N_pad - N))

    out_t = _make_sc_decode(N_pad, A, P, NC, NW, n_iters)(table, acts)
    return out_t[:, :N].T.reshape(-1)


# speedup vs baseline: 1.4530x; 1.4530x over previous
import functools

import jax
import jax.numpy as jnp
from jax import lax
from jax.experimental import pallas as pl
from jax.experimental.pallas import tpu as pltpu
from jax.experimental.pallas import tpu_sc as plsc

_BLK = 128


def _make_sc_decode(N, N_pad, A, P, NC, NW, n_iters):
    A_pad = ((A + 15) // 16) * 16
    mesh = plsc.VectorSubcoreMesh(core_axis_name="c", subcore_axis_name="s")
    tile_words = P * _BLK

    @functools.partial(
        pl.kernel,
        mesh=mesh,
        out_type=jax.ShapeDtypeStruct((N * P,), jnp.float32),
        scratch_types=[
            pltpu.VMEM((_BLK,), jnp.int32),
            pltpu.VMEM((A_pad,), jnp.int32),
            pltpu.VMEM((A, P, _BLK), jnp.float32),
            pltpu.VMEM((_BLK * P,), jnp.float32),
            pltpu.SemaphoreType.DMA,
            pltpu.SemaphoreType.DMA,
        ],
        compiler_params=pltpu.CompilerParams(
            use_tc_tiling_on_sc=True, needs_layout_passes=False
        ),
    )
    def k(table_hbm, acts_hbm, out_hbm, acts_v, present_v, buf,
          rows_v, sem, sem_out):
        wid = lax.axis_index("s") * NC + lax.axis_index("c")
        lane = lax.iota(jnp.int32, 16)
        lane8 = lane * P
        zeros16 = jnp.zeros((16,), jnp.int32)
        ones16 = jnp.ones((16,), jnp.int32)

        def do_block(kb, _):
            b = pl.multiple_of((kb * NW + wid) * _BLK, _BLK)

            @pl.when(b < N_pad)
            def _process():
                pltpu.sync_copy(acts_hbm.at[pl.ds(b, _BLK)], acts_v)
                for ch in range(A_pad // 16):
                    present_v[pl.ds(ch * 16, 16)] = zeros16
                for ch in range(_BLK // 16):
                    a16 = acts_v[pl.ds(ch * 16, 16)]
                    plsc.store_scatter(present_v, [a16], ones16)
                pres = [
                    present_v[pl.ds(ch * 16, 16)] for ch in range(A_pad // 16)
                ]

                for a in range(A):
                    @pl.when(pres[a // 16][a % 16] != 0)
                    def _go(a=a):
                        pltpu.async_copy(
                            table_hbm.at[a, :, pl.ds(b, _BLK)],
                            buf.at[a],
                            sem,
                        )

                acc = pres[0]
                for ch in range(1, A_pad // 16):
                    acc = acc + pres[ch]
                n_tiles = jnp.sum(acc)

                def drain(_i, _c):
                    pltpu.make_async_copy(
                        table_hbm.at[0, :, pl.ds(b, _BLK)],
                        buf.at[0],
                        sem,
                    ).wait()
                    return _c

                lax.fori_loop(0, n_tiles, drain, 0)

                for ch in range(_BLK // 16):
                    a16 = acts_v[pl.ds(ch * 16, 16)]
                    l16 = ch * 16 + lane
                    for p in range(P):
                        vals = plsc.load_gather(
                            buf, [a16, jnp.full((16,), p, jnp.int32), l16]
                        )
                        plsc.store_scatter(
                            rows_v, [lane8 + (ch * 16 * P + p)], vals
                        )

                @pl.when(b + _BLK <= N)
                def _full():
                    pltpu.async_copy(
                        rows_v, out_hbm.at[pl.ds(b * P, _BLK * P)], sem_out
                    ).wait()

                @pl.when(b + _BLK > N)
                def _tail():
                    pltpu.async_copy(
                        rows_v.at[pl.ds(0, (N % _BLK) * P)],
                        out_hbm.at[pl.ds(b * P, (N % _BLK) * P)],
                        sem_out,
                    ).wait()

            return _

        lax.fori_loop(0, n_iters, do_block, 0)

    return k


def kernel(abs_actions, decode_map):
    N, A, P = decode_map.shape
    info = plsc.get_sparse_core_info()
    NC = info.num_cores
    NW = NC * info.num_subcores

    N_pad = ((N + _BLK - 1) // _BLK) * _BLK
    NB = N_pad // _BLK
    n_iters = (NB + NW - 1) // NW

    table = decode_map.transpose(1, 2, 0)
    acts = jnp.pad(abs_actions.astype(jnp.int32), (0, N_pad - N))

    return _make_sc_decode(N, N_pad, A, P, NC, NW, n_iters)(table, acts)
